# baseline (speedup 1.0000x reference)
from math import exp, log

import numpy as np
import jax
import jax.numpy as jnp
from jax import lax
from jax.experimental import pallas as pl
from jax.experimental.pallas import tpu as pltpu
from jax.experimental.pallas import tpu_sc as plsc

N_LEVELS = 16
F_PER_LEVEL = 2
LOG2_T = 19
T = 1 << LOG2_T
BASE_RES = 16
MAX_RES = 2048
SCALE = exp((log(MAX_RES) - log(BASE_RES)) / (N_LEVELS - 1))
RES = [int(np.floor(BASE_RES * (SCALE ** l))) for l in range(N_LEVELS)]
DENSE = [(r + 1) ** 3 <= T for r in RES]
P1 = np.int32(-1640531535)
P2 = np.int32(805459861)
HMASK = np.int32(T - 1)
CORNERS = [(i, j, k) for i in (0, 1) for j in (0, 1) for k in (0, 1)]

N_POINTS = 524288
NC, NS = 2, 16
NW = NC * NS
NPT = N_POINTS // NW
C = 512
G = C // 16
NCH = NPT // C
NIDX = 8 * C
HI16 = np.int32(-65536)

N_LOCAL = 3
LOC_PAD = [((RES[l] + 1) ** 3 + 7) // 8 * 8 for l in range(N_LOCAL)]

PHASES = [[3, 4]] + [[l] for l in range(5, N_LEVELS)]


def _lv_words(l):
    return ((RES[l] + 1) ** 3 + 7) // 8 * 8 if DENSE[l] else T


SP_OFF = {}
SP_WORDS = 0
for _ph in PHASES:
    _o = 0
    for _l in _ph:
        SP_OFF[_l] = _o
        _o += _lv_words(_l)
    SP_WORDS = max(SP_WORDS, _o)


def _body(x_hbm, tab_hbm, out_hbm, xbuf, frac0, frac1, idxbuf0, idxbuf1,
          gath0, gath1, outc, tabl0, tabl1, tabl2, shared, sem):
    cid = lax.axis_index("c")
    sid = lax.axis_index("s")
    wid = sid * NC + cid
    lane = lax.iota(jnp.int32, 16)
    lane3 = lane * 3

    pltpu.sync_copy(tab_hbm.at[pl.ds(0, LOC_PAD[0])], tabl0)
    pltpu.sync_copy(tab_hbm.at[pl.ds(T, LOC_PAD[1])], tabl1)
    pltpu.sync_copy(tab_hbm.at[pl.ds(2 * T, LOC_PAD[2])], tabl2)

    def make_chunk(levels, with_local):
        r0 = 0 if with_local else 2 * levels[0]
        nr = 2 * (levels[-1] + 1) - r0

        def chunk_body(ch, carry):
            base = wid * NPT + ch * C
            pltpu.sync_copy(x_hbm.at[pl.ds(base * 3, C * 3)], xbuf)

            def make_p1(l):
                res = RES[l]
                resf = np.float32(res)
                idxbuf = idxbuf0 if l % 2 == 0 else idxbuf1
                fracbuf = frac0 if l % 2 == 0 else frac1
                off = np.int32(SP_OFF[l])

                def p1(g, c1):
                    b16 = g * 16
                    xi = b16 * 3 + lane3
                    px = plsc.load_gather(xbuf, [xi])
                    py = plsc.load_gather(xbuf, [xi + 1])
                    pz = plsc.load_gather(xbuf, [xi + 2])
                    posx = px * resf
                    posy = py * resf
                    posz = pz * resf
                    ix = posx.astype(jnp.int32)
                    iy = posy.astype(jnp.int32)
                    iz = posz.astype(jnp.int32)
                    fracbuf[0, pl.ds(b16, 16)] = posx - ix.astype(jnp.float32)
                    fracbuf[1, pl.ds(b16, 16)] = posy - iy.astype(jnp.float32)
                    fracbuf[2, pl.ds(b16, 16)] = posz - iz.astype(jnp.float32)
                    for ci, (i, j, k) in enumerate(CORNERS):
                        cx = ix + i if i else ix
                        cy = iy + j if j else iy
                        cz = iz + k if k else iz
                        if DENSE[l]:
                            s = np.int32(res + 1)
                            s2 = np.int32((res + 1) * (res + 1))
                            idx = cx + cy * s + cz * s2
                        else:
                            idx = (cx ^ (cy * P1) ^ (cz * P2)) & HMASK
                        idxbuf[pl.ds(ci * C + b16, 16)] = idx + off
                    return c1

                return p1

            def make_p2(l):
                gath = gath0 if l % 2 == 0 else gath1
                fracbuf = frac0 if l % 2 == 0 else frac1

                def p2(g, c2):
                    b16 = g * 16
                    fx = fracbuf[0, pl.ds(b16, 16)]
                    fy = fracbuf[1, pl.ds(b16, 16)]
                    fz = fracbuf[2, pl.ds(b16, 16)]
                    gx = 1.0 - fx
                    gy = 1.0 - fy
                    gz = 1.0 - fz
                    wyz = (gy * gz, gy * fz, fy * gz, fy * fz)
                    acc0 = jnp.zeros((16,), jnp.float32)
                    acc1 = jnp.zeros((16,), jnp.float32)
                    for ci, (i, j, k) in enumerate(CORNERS):
                        v = gath[pl.ds(ci * C + b16, 16)]
                        f0 = plsc.bitcast(lax.shift_left(v, 16), jnp.float32)
                        f1 = plsc.bitcast(v & HI16, jnp.float32)
                        w = (fx if i else gx) * wyz[2 * j + k]
                        acc0 = acc0 + w * f0
                        acc1 = acc1 + w * f1
                    outc[2 * l, pl.ds(b16, 16)] = acc0
                    outc[2 * l + 1, pl.ds(b16, 16)] = acc1
                    return c2

                return p2

            def make_local(l, tabl):
                res = RES[l]
                resf = np.float32(res)

                def lp(g, c1):
                    b16 = g * 16
                    xi = b16 * 3 + lane3
                    px = plsc.load_gather(xbuf, [xi])
                    py = plsc.load_gather(xbuf, [xi + 1])
                    pz = plsc.load_gather(xbuf, [xi + 2])
                    posx = px * resf
                    posy = py * resf
                    posz = pz * resf
                    ix = posx.astype(jnp.int32)
                    iy = posy.astype(jnp.int32)
                    iz = posz.astype(jnp.int32)
                    fx = posx - ix.astype(jnp.float32)
                    fy = posy - iy.astype(jnp.float32)
                    fz = posz - iz.astype(jnp.float32)
                    gx = 1.0 - fx
                    gy = 1.0 - fy
                    gz = 1.0 - fz
                    wyz = (gy * gz, gy * fz, fy * gz, fy * fz)
                    acc0 = jnp.zeros((16,), jnp.float32)
                    acc1 = jnp.zeros((16,), jnp.float32)
                    s = np.int32(res + 1)
                    s2 = np.int32((res + 1) * (res + 1))
                    for ci, (i, j, k) in enumerate(CORNERS):
                        cx = ix + i if i else ix
                        cy = iy + j if j else iy
                        cz = iz + k if k else iz
                        idx = cx + cy * s + cz * s2
                        v = plsc.load_gather(tabl, [idx])
                        f0 = plsc.bitcast(lax.shift_left(v, 16), jnp.float32)
                        f1 = plsc.bitcast(v & HI16, jnp.float32)
                        w = (fx if i else gx) * wyz[2 * j + k]
                        acc0 = acc0 + w * f0
                        acc1 = acc1 + w * f1
                    outc[2 * l, pl.ds(b16, 16)] = acc0
                    outc[2 * l + 1, pl.ds(b16, 16)] = acc1
                    return c1

                return lp

            def fire(l):
                ib = idxbuf0 if l % 2 == 0 else idxbuf1
                gb = gath0 if l % 2 == 0 else gath1
                return pltpu.async_copy(shared.at[ib], gb, sem)

            first = levels[0]
            lax.fori_loop(0, G, make_p1(first), 0, unroll=2)
            h = {first: fire(first)}
            if with_local:
                for loc, tl in ((0, tabl0), (1, tabl1), (2, tabl2)):
                    lax.fori_loop(0, G, make_local(loc, tl), 0, unroll=2)
            for pos in range(1, len(levels) + 1):
                if pos < len(levels):
                    l = levels[pos]
                    lax.fori_loop(0, G, make_p1(l), 0, unroll=2)
                lp = levels[pos - 1]
                h.pop(lp).wait()
                if pos < len(levels):
                    h[levels[pos]] = fire(levels[pos])
                lax.fori_loop(0, G, make_p2(lp), 0, unroll=2)
            pltpu.sync_copy(outc.at[pl.ds(r0, nr)],
                            out_hbm.at[pl.ds(r0, nr), pl.ds(base, C)])
            return carry

        return chunk_body

    for pi, ph in enumerate(PHASES):
        @pl.when(sid == 0)
        def _():
            for l in ph:
                pltpu.sync_copy(
                    tab_hbm.at[pl.ds(l * T, _lv_words(l))],
                    shared.at[pl.ds(SP_OFF[l], _lv_words(l))],
                )

        plsc.subcore_barrier()
        lax.fori_loop(0, NCH, make_chunk(ph, pi == 0), 0, unroll=False)
        plsc.subcore_barrier()


_mesh = plsc.VectorSubcoreMesh(
    core_axis_name="c", subcore_axis_name="s", num_cores=2, num_subcores=16
)

_call = pl.kernel(
    _body,
    out_type=jax.ShapeDtypeStruct((N_LEVELS * F_PER_LEVEL, N_POINTS), jnp.float32),
    mesh=_mesh,
    scratch_types=[
        pltpu.VMEM((3 * C,), jnp.float32),
        pltpu.VMEM((3, C), jnp.float32),
        pltpu.VMEM((3, C), jnp.float32),
        pltpu.VMEM((NIDX,), jnp.int32),
        pltpu.VMEM((NIDX,), jnp.int32),
        pltpu.VMEM((NIDX,), jnp.int32),
        pltpu.VMEM((NIDX,), jnp.int32),
        pltpu.VMEM((N_LEVELS * F_PER_LEVEL, C), jnp.float32),
        pltpu.VMEM((LOC_PAD[0],), jnp.int32),
        pltpu.VMEM((LOC_PAD[1],), jnp.int32),
        pltpu.VMEM((LOC_PAD[2],), jnp.int32),
        pltpu.VMEM_SHARED((SP_WORDS,), jnp.int32),
        pltpu.SemaphoreType.DMA,
    ],
    compiler_params=pltpu.CompilerParams(
        needs_layout_passes=False, use_tc_tiling_on_sc=False
    ),
)


def kernel(x, table):
    xf = x.reshape(-1)
    tb = table.astype(jnp.bfloat16)
    u0 = lax.bitcast_convert_type(tb[..., 0], jnp.uint16).astype(jnp.uint32)
    u1 = lax.bitcast_convert_type(tb[..., 1], jnp.uint16).astype(jnp.uint32)
    tf = lax.bitcast_convert_type(u0 | (u1 << 16), jnp.int32).reshape(-1)
    out = _call(xf, tf)
    return out.T


# speedup vs baseline: 1.0428x; 1.0428x over previous
from math import exp, log

import numpy as np
import jax
import jax.numpy as jnp
from jax import lax
from jax.experimental import pallas as pl
from jax.experimental.pallas import tpu as pltpu
from jax.experimental.pallas import tpu_sc as plsc

N_LEVELS = 16
F_PER_LEVEL = 2
LOG2_T = 19
T = 1 << LOG2_T
BASE_RES = 16
MAX_RES = 2048
SCALE = exp((log(MAX_RES) - log(BASE_RES)) / (N_LEVELS - 1))
RES = [int(np.floor(BASE_RES * (SCALE ** l))) for l in range(N_LEVELS)]
DENSE = [(r + 1) ** 3 <= T for r in RES]
P1 = np.int32(-1640531535)
P2 = np.int32(805459861)
HMASK = np.int32(T - 1)
CORNERS = [(i, j, k) for i in (0, 1) for j in (0, 1) for k in (0, 1)]

N_POINTS = 524288
NC, NS = 2, 16
NW = NC * NS
NPT = N_POINTS // NW
C = 512
G = C // 16
NCH = NPT // C
NIDX = 8 * C
HI16 = np.int32(-65536)

N_LOCAL = 3
LOC_PAD = [((RES[l] + 1) ** 3 + 7) // 8 * 8 for l in range(N_LOCAL)]

PHASES = [[3, 4]] + [[l] for l in range(5, N_LEVELS)]


def _lv_words(l):
    return ((RES[l] + 1) ** 3 + 7) // 8 * 8 if DENSE[l] else T


SP_OFF = {}
SP_WORDS = 0
for _ph in PHASES:
    _o = 0
    for _l in _ph:
        SP_OFF[_l] = _o
        _o += _lv_words(_l)
    SP_WORDS = max(SP_WORDS, _o)


def _body(x_hbm, tab_hbm, out_hbm, xbuf, frac0, frac1, idxbuf0, idxbuf1,
          gath0, gath1, outc, tabl0, tabl1, tabl2, shared, sem):
    cid = lax.axis_index("c")
    sid = lax.axis_index("s")
    wid = sid * NC + cid
    lane = lax.iota(jnp.int32, 16)
    lane3 = lane * 3

    pltpu.sync_copy(tab_hbm.at[pl.ds(0, LOC_PAD[0])], tabl0)
    pltpu.sync_copy(tab_hbm.at[pl.ds(T, LOC_PAD[1])], tabl1)
    pltpu.sync_copy(tab_hbm.at[pl.ds(2 * T, LOC_PAD[2])], tabl2)

    def make_chunk(levels, with_local):
        r0 = 0 if with_local else 2 * levels[0]
        nr = 2 * (levels[-1] + 1) - r0

        def chunk_body(ch, carry):
            base = wid * NPT + ch * C
            pltpu.sync_copy(x_hbm.at[pl.ds(base * 3, C * 3)], xbuf)

            def make_p1(l):
                res = RES[l]
                resf = np.float32(res)
                idxbuf = idxbuf0 if l % 2 == 0 else idxbuf1
                fracbuf = frac0 if l % 2 == 0 else frac1
                off = np.int32(SP_OFF[l])

                def p1(g, c1):
                    b16 = g * 16
                    xi = b16 * 3 + lane3
                    px = plsc.load_gather(xbuf, [xi])
                    py = plsc.load_gather(xbuf, [xi + 1])
                    pz = plsc.load_gather(xbuf, [xi + 2])
                    posx = px * resf
                    posy = py * resf
                    posz = pz * resf
                    ix = posx.astype(jnp.int32)
                    iy = posy.astype(jnp.int32)
                    iz = posz.astype(jnp.int32)
                    fracbuf[0, pl.ds(b16, 16)] = posx - ix.astype(jnp.float32)
                    fracbuf[1, pl.ds(b16, 16)] = posy - iy.astype(jnp.float32)
                    fracbuf[2, pl.ds(b16, 16)] = posz - iz.astype(jnp.float32)
                    for ci, (i, j, k) in enumerate(CORNERS):
                        cx = ix + i if i else ix
                        cy = iy + j if j else iy
                        cz = iz + k if k else iz
                        if DENSE[l]:
                            s = np.int32(res + 1)
                            s2 = np.int32((res + 1) * (res + 1))
                            idx = cx + cy * s + cz * s2
                        else:
                            idx = (cx ^ (cy * P1) ^ (cz * P2)) & HMASK
                        idxbuf[pl.ds(ci * C + b16, 16)] = idx + off
                    return c1

                return p1

            def make_p2(l):
                gath = gath0 if l % 2 == 0 else gath1
                fracbuf = frac0 if l % 2 == 0 else frac1

                def p2(g, c2):
                    b16 = g * 16
                    fx = fracbuf[0, pl.ds(b16, 16)]
                    fy = fracbuf[1, pl.ds(b16, 16)]
                    fz = fracbuf[2, pl.ds(b16, 16)]
                    gx = 1.0 - fx
                    gy = 1.0 - fy
                    gz = 1.0 - fz
                    wyz = (gy * gz, gy * fz, fy * gz, fy * fz)
                    acc0 = jnp.zeros((16,), jnp.float32)
                    acc1 = jnp.zeros((16,), jnp.float32)
                    for ci, (i, j, k) in enumerate(CORNERS):
                        v = gath[pl.ds(ci * C + b16, 16)]
                        f0 = plsc.bitcast(lax.shift_left(v, 16), jnp.float32)
                        f1 = plsc.bitcast(v & HI16, jnp.float32)
                        w = (fx if i else gx) * wyz[2 * j + k]
                        acc0 = acc0 + w * f0
                        acc1 = acc1 + w * f1
                    outc[2 * l, pl.ds(b16, 16)] = acc0
                    outc[2 * l + 1, pl.ds(b16, 16)] = acc1
                    return c2

                return p2

            def make_local(l, tabl):
                res = RES[l]
                resf = np.float32(res)

                def lp(g, c1):
                    b16 = g * 16
                    xi = b16 * 3 + lane3
                    px = plsc.load_gather(xbuf, [xi])
                    py = plsc.load_gather(xbuf, [xi + 1])
                    pz = plsc.load_gather(xbuf, [xi + 2])
                    posx = px * resf
                    posy = py * resf
                    posz = pz * resf
                    ix = posx.astype(jnp.int32)
                    iy = posy.astype(jnp.int32)
                    iz = posz.astype(jnp.int32)
                    fx = posx - ix.astype(jnp.float32)
                    fy = posy - iy.astype(jnp.float32)
                    fz = posz - iz.astype(jnp.float32)
                    gx = 1.0 - fx
                    gy = 1.0 - fy
                    gz = 1.0 - fz
                    wyz = (gy * gz, gy * fz, fy * gz, fy * fz)
                    acc0 = jnp.zeros((16,), jnp.float32)
                    acc1 = jnp.zeros((16,), jnp.float32)
                    s = np.int32(res + 1)
                    s2 = np.int32((res + 1) * (res + 1))
                    for ci, (i, j, k) in enumerate(CORNERS):
                        cx = ix + i if i else ix
                        cy = iy + j if j else iy
                        cz = iz + k if k else iz
                        idx = cx + cy * s + cz * s2
                        v = plsc.load_gather(tabl, [idx])
                        f0 = plsc.bitcast(lax.shift_left(v, 16), jnp.float32)
                        f1 = plsc.bitcast(v & HI16, jnp.float32)
                        w = (fx if i else gx) * wyz[2 * j + k]
                        acc0 = acc0 + w * f0
                        acc1 = acc1 + w * f1
                    outc[2 * l, pl.ds(b16, 16)] = acc0
                    outc[2 * l + 1, pl.ds(b16, 16)] = acc1
                    return c1

                return lp

            def fire(l):
                ib = idxbuf0 if l % 2 == 0 else idxbuf1
                gb = gath0 if l % 2 == 0 else gath1
                return pltpu.async_copy(shared.at[ib], gb, sem)

            first = levels[0]
            lax.fori_loop(0, G, make_p1(first), 0, unroll=False)
            h = {first: fire(first)}
            if with_local:
                for loc, tl in ((0, tabl0), (1, tabl1), (2, tabl2)):
                    lax.fori_loop(0, G, make_local(loc, tl), 0, unroll=False)
            for pos in range(1, len(levels) + 1):
                if pos < len(levels):
                    l = levels[pos]
                    lax.fori_loop(0, G, make_p1(l), 0, unroll=False)
                lp = levels[pos - 1]
                h.pop(lp).wait()
                if pos < len(levels):
                    h[levels[pos]] = fire(levels[pos])
                lax.fori_loop(0, G, make_p2(lp), 0, unroll=False)
            pltpu.sync_copy(outc.at[pl.ds(r0, nr)],
                            out_hbm.at[pl.ds(r0, nr), pl.ds(base, C)])
            return carry

        return chunk_body

    for pi, ph in enumerate(PHASES):
        @pl.when(sid == 0)
        def _():
            for l in ph:
                pltpu.sync_copy(
                    tab_hbm.at[pl.ds(l * T, _lv_words(l))],
                    shared.at[pl.ds(SP_OFF[l], _lv_words(l))],
                )

        plsc.subcore_barrier()
        lax.fori_loop(0, NCH, make_chunk(ph, pi == 0), 0, unroll=False)
        plsc.subcore_barrier()


_mesh = plsc.VectorSubcoreMesh(
    core_axis_name="c", subcore_axis_name="s", num_cores=2, num_subcores=16
)

_call = pl.kernel(
    _body,
    out_type=jax.ShapeDtypeStruct((N_LEVELS * F_PER_LEVEL, N_POINTS), jnp.float32),
    mesh=_mesh,
    scratch_types=[
        pltpu.VMEM((3 * C,), jnp.float32),
        pltpu.VMEM((3, C), jnp.float32),
        pltpu.VMEM((3, C), jnp.float32),
        pltpu.VMEM((NIDX,), jnp.int32),
        pltpu.VMEM((NIDX,), jnp.int32),
        pltpu.VMEM((NIDX,), jnp.int32),
        pltpu.VMEM((NIDX,), jnp.int32),
        pltpu.VMEM((N_LEVELS * F_PER_LEVEL, C), jnp.float32),
        pltpu.VMEM((LOC_PAD[0],), jnp.int32),
        pltpu.VMEM((LOC_PAD[1],), jnp.int32),
        pltpu.VMEM((LOC_PAD[2],), jnp.int32),
        pltpu.VMEM_SHARED((SP_WORDS,), jnp.int32),
        pltpu.SemaphoreType.DMA,
    ],
    compiler_params=pltpu.CompilerParams(
        needs_layout_passes=False, use_tc_tiling_on_sc=False
    ),
)


def kernel(x, table):
    xf = x.reshape(-1)
    tb = table.astype(jnp.bfloat16)
    u0 = lax.bitcast_convert_type(tb[..., 0], jnp.uint16).astype(jnp.uint32)
    u1 = lax.bitcast_convert_type(tb[..., 1], jnp.uint16).astype(jnp.uint32)
    tf = lax.bitcast_convert_type(u0 | (u1 << 16), jnp.int32).reshape(-1)
    out = _call(xf, tf)
    return out.T
